# Initial kernel scaffold; baseline (speedup 1.0000x reference)
import jax
import jax.numpy as jnp
from jax.experimental import pallas as pl


def kernel(x, feat_same, feat_anti, senders_same, receivers_same, senders_anti, receivers_anti, W_u_same, b_u_same, W_u_anti, b_u_anti, W_w_same, b_w_same, W_w_anti, b_w_anti, W_h_same, b_h_same, W_h_anti, b_h_anti, W_g, b_g):
    raise NotImplementedError("write your pallas kernel here")


# speedup vs baseline: 2.6773x; 2.6773x over previous
import functools

import jax
import jax.numpy as jnp
from jax import lax
from jax.experimental import pallas as pl
from jax.experimental.pallas import tpu as pltpu
from jax.experimental.pallas import tpu_sc as plsc

N = 10000
D = 128
E = 320000

NUM_TILES = 16
EDGES_PER_TILE = E // NUM_TILES
CHUNK = 80
NCHUNKS = EDGES_PER_TILE // CHUNK
ROWS_PER_TILE = 624
ZCHUNK = 208
NZ = ROWS_PER_TILE // ZCHUNK
TAIL_ROWS = N - NUM_TILES * ROWS_PER_TILE

_F32 = jnp.float32


def _hx_body(x_ref, ws_ref, bs_ref, wa_ref, ba_ref, hs_ref, ha_ref):
    xv = x_ref[...]
    hs_ref[...] = jnp.tanh(
        jnp.dot(xv, ws_ref[...], preferred_element_type=_F32) + bs_ref[...])
    ha_ref[...] = jnp.tanh(
        jnp.dot(xv, wa_ref[...], preferred_element_type=_F32) + ba_ref[...])


_BN = 2000

_hx_call = pl.pallas_call(
    _hx_body,
    grid=(N // _BN,),
    in_specs=[
        pl.BlockSpec((_BN, D), lambda i: (i, 0)),
        pl.BlockSpec((D, D), lambda i: (0, 0)),
        pl.BlockSpec((1, D), lambda i: (0, 0)),
        pl.BlockSpec((D, D), lambda i: (0, 0)),
        pl.BlockSpec((1, D), lambda i: (0, 0)),
    ],
    out_specs=[
        pl.BlockSpec((_BN, D), lambda i: (i, 0)),
        pl.BlockSpec((_BN, D), lambda i: (i, 0)),
    ],
    out_shape=[
        jax.ShapeDtypeStruct((N, D), _F32),
        jax.ShapeDtypeStruct((N, D), _F32),
    ],
)


def _we_body(f_ref, wu_ref, bu_ref, ww_ref, bw_ref, o_ref):
    fb = f_ref[...]
    f = jnp.tanh(
        jnp.dot(fb, wu_ref[...], preferred_element_type=_F32) + bu_ref[...]) + fb
    o_ref[...] = jnp.tanh(
        jnp.dot(f, ww_ref[...], preferred_element_type=_F32) + bw_ref[...])


_BE = 5000

_we_call = pl.pallas_call(
    _we_body,
    grid=(E // _BE,),
    in_specs=[
        pl.BlockSpec((_BE, D), lambda i: (i, 0)),
        pl.BlockSpec((D, D), lambda i: (0, 0)),
        pl.BlockSpec((1, D), lambda i: (0, 0)),
        pl.BlockSpec((D, D), lambda i: (0, 0)),
        pl.BlockSpec((1, D), lambda i: (0, 0)),
    ],
    out_specs=pl.BlockSpec((_BE, D), lambda i: (i, 0)),
    out_shape=jax.ShapeDtypeStruct((E, D), _F32),
)


def _upd_body(x_ref, zs_ref, za_ref, wg_ref, bg_ref, o_ref):
    xv = x_ref[...]
    acc = jnp.dot(xv, wg_ref[0:D, :], preferred_element_type=_F32)
    acc = acc + jnp.dot(zs_ref[...], wg_ref[D:2 * D, :],
                        preferred_element_type=_F32)
    acc = acc + jnp.dot(za_ref[...], wg_ref[2 * D:3 * D, :],
                        preferred_element_type=_F32)
    o_ref[...] = xv + jnp.tanh(acc + bg_ref[...])


_upd_call = pl.pallas_call(
    _upd_body,
    grid=(N // _BN,),
    in_specs=[
        pl.BlockSpec((_BN, D), lambda i: (i, 0)),
        pl.BlockSpec((_BN, D), lambda i: (i, 0)),
        pl.BlockSpec((_BN, D), lambda i: (i, 0)),
        pl.BlockSpec((3 * D, D), lambda i: (0, 0)),
        pl.BlockSpec((1, D), lambda i: (0, 0)),
    ],
    out_specs=pl.BlockSpec((_BN, D), lambda i: (i, 0)),
    out_shape=jax.ShapeDtypeStruct((N, D), _F32),
)


@functools.partial(
    pl.kernel,
    out_type=(
        jax.ShapeDtypeStruct((N, D), _F32),
        jax.ShapeDtypeStruct((N, D), _F32),
    ),
    mesh=plsc.VectorSubcoreMesh(core_axis_name="c", subcore_axis_name="s"),
    scratch_types=[
        pltpu.VMEM((CHUNK,), jnp.int32),
        pltpu.VMEM((CHUNK,), jnp.int32),
        pltpu.VMEM((CHUNK, D), _F32),
        pltpu.VMEM((CHUNK, D), _F32),
        pltpu.VMEM((ZCHUNK, D), _F32),
        pltpu.VMEM_SHARED((N, D), _F32),
        pltpu.SemaphoreType.DMA,
    ],
)
def _sc_aggregate(hx_s, we_s, send_s, recv_s, hx_a, we_a, send_a, recv_a,
                  z_s, z_a, sidx, ridx, hx_rows, we_rows, zbuf, z_sh, sem):
    c = lax.axis_index("c")
    s = lax.axis_index("s")

    zeros16 = jnp.zeros((16,), _F32)

    def _zrow(r, carry):
        for g in range(D // 16):
            zbuf[r, pl.ds(g * 16, 16)] = zeros16
        return carry

    lax.fori_loop(0, ZCHUNK, _zrow, 0)
    for j in range(NZ):
        pltpu.sync_copy(
            zbuf, z_sh.at[pl.ds(s * ROWS_PER_TILE + j * ZCHUNK, ZCHUNK)])

    @pl.when(s == 0)
    def _():
        pltpu.sync_copy(zbuf.at[pl.ds(0, TAIL_ROWS)],
                        z_sh.at[pl.ds(NUM_TILES * ROWS_PER_TILE, TAIL_ROWS)])

    plsc.subcore_barrier()

    def _process(hx_hbm, we_hbm, send_hbm, recv_hbm, z_hbm):
        def _chunk(k, carry):
            base = s * EDGES_PER_TILE + k * CHUNK
            pltpu.sync_copy(send_hbm.at[pl.ds(base, CHUNK)], sidx)
            pltpu.sync_copy(recv_hbm.at[pl.ds(base, CHUNK)], ridx)
            gather = pltpu.async_copy(hx_hbm.at[sidx], hx_rows, sem)
            pltpu.sync_copy(we_hbm.at[pl.ds(base, CHUNK)], we_rows)
            gather.wait()

            def _mul(r, cc):
                for g in range(D // 16):
                    sl = pl.ds(g * 16, 16)
                    we_rows[r, sl] = we_rows[r, sl] * hx_rows[r, sl]
                return cc

            lax.fori_loop(0, CHUNK, _mul, 0)
            pltpu.sync_copy(we_rows, z_sh.at[ridx], add=True)
            return carry

        lax.fori_loop(0, NCHUNKS, _chunk, 0)
        plsc.subcore_barrier()
        for j in range(NZ):
            sl = pl.ds(s * ROWS_PER_TILE + j * ZCHUNK, ZCHUNK)
            pltpu.sync_copy(z_sh.at[sl], z_hbm.at[sl])

        @pl.when(s == 0)
        def _():
            sl = pl.ds(NUM_TILES * ROWS_PER_TILE, TAIL_ROWS)
            pltpu.sync_copy(z_sh.at[sl], z_hbm.at[sl])

    @pl.when(c == 0)
    def _():
        _process(hx_s, we_s, send_s, recv_s, z_s)

    @pl.when(c == 1)
    def _():
        _process(hx_a, we_a, send_a, recv_a, z_a)


def kernel(x, feat_same, feat_anti, senders_same, receivers_same, senders_anti,
           receivers_anti, W_u_same, b_u_same, W_u_anti, b_u_anti, W_w_same,
           b_w_same, W_w_anti, b_w_anti, W_h_same, b_h_same, W_h_anti,
           b_h_anti, W_g, b_g):
    r = lambda b: b.reshape(1, D)
    hx_s, hx_a = _hx_call(x, W_h_same, r(b_h_same), W_h_anti, r(b_h_anti))
    we_s = _we_call(feat_same, W_u_same, r(b_u_same), W_w_same, r(b_w_same))
    we_a = _we_call(feat_anti, W_u_anti, r(b_u_anti), W_w_anti, r(b_w_anti))
    i32 = jnp.int32
    z_s, z_a = _sc_aggregate(
        hx_s, we_s, senders_same.astype(i32), receivers_same.astype(i32),
        hx_a, we_a, senders_anti.astype(i32), receivers_anti.astype(i32))
    return _upd_call(x, z_s, z_a, W_g, r(b_g))


# speedup vs baseline: 3.5952x; 1.3428x over previous
import functools

import jax
import jax.numpy as jnp
from jax import lax
from jax.experimental import pallas as pl
from jax.experimental.pallas import tpu as pltpu
from jax.experimental.pallas import tpu_sc as plsc

N = 10000
D = 128
E = 320000

NUM_TILES = 16
EDGES_PER_TILE = E // NUM_TILES
CHUNK = 80
NFULL = EDGES_PER_TILE // CHUNK
NPAIR = NFULL // 2
ROWS_PER_TILE = 624
ZCHUNK = 16
NZ = ROWS_PER_TILE // ZCHUNK
TAIL_ROWS = N - NUM_TILES * ROWS_PER_TILE

_F32 = jnp.float32


def _hx_body(x_ref, ws_ref, bs_ref, wa_ref, ba_ref, hs_ref, ha_ref):
    xv = x_ref[...]
    hs_ref[...] = jnp.tanh(
        jnp.dot(xv, ws_ref[...], preferred_element_type=_F32) + bs_ref[...])
    ha_ref[...] = jnp.tanh(
        jnp.dot(xv, wa_ref[...], preferred_element_type=_F32) + ba_ref[...])


_BN = 2000

_hx_call = pl.pallas_call(
    _hx_body,
    grid=(N // _BN,),
    in_specs=[
        pl.BlockSpec((_BN, D), lambda i: (i, 0)),
        pl.BlockSpec((D, D), lambda i: (0, 0)),
        pl.BlockSpec((1, D), lambda i: (0, 0)),
        pl.BlockSpec((D, D), lambda i: (0, 0)),
        pl.BlockSpec((1, D), lambda i: (0, 0)),
    ],
    out_specs=[
        pl.BlockSpec((_BN, D), lambda i: (i, 0)),
        pl.BlockSpec((_BN, D), lambda i: (i, 0)),
    ],
    out_shape=[
        jax.ShapeDtypeStruct((N, D), _F32),
        jax.ShapeDtypeStruct((N, D), _F32),
    ],
)


def _we_body(f_ref, wu_ref, bu_ref, ww_ref, bw_ref, o_ref):
    fb = f_ref[...]
    f = jnp.tanh(
        jnp.dot(fb, wu_ref[...], preferred_element_type=_F32) + bu_ref[...]) + fb
    o_ref[...] = jnp.tanh(
        jnp.dot(f, ww_ref[...], preferred_element_type=_F32) + bw_ref[...])


_BE = 5000

_we_call = pl.pallas_call(
    _we_body,
    grid=(E // _BE,),
    in_specs=[
        pl.BlockSpec((_BE, D), lambda i: (i, 0)),
        pl.BlockSpec((D, D), lambda i: (0, 0)),
        pl.BlockSpec((1, D), lambda i: (0, 0)),
        pl.BlockSpec((D, D), lambda i: (0, 0)),
        pl.BlockSpec((1, D), lambda i: (0, 0)),
    ],
    out_specs=pl.BlockSpec((_BE, D), lambda i: (i, 0)),
    out_shape=jax.ShapeDtypeStruct((E, D), _F32),
)


def _upd_body(x_ref, zs_ref, za_ref, wg_ref, bg_ref, o_ref):
    xv = x_ref[...]
    acc = jnp.dot(xv, wg_ref[0:D, :], preferred_element_type=_F32)
    acc = acc + jnp.dot(zs_ref[...], wg_ref[D:2 * D, :],
                        preferred_element_type=_F32)
    acc = acc + jnp.dot(za_ref[...], wg_ref[2 * D:3 * D, :],
                        preferred_element_type=_F32)
    o_ref[...] = xv + jnp.tanh(acc + bg_ref[...])


_upd_call = pl.pallas_call(
    _upd_body,
    grid=(N // _BN,),
    in_specs=[
        pl.BlockSpec((_BN, D), lambda i: (i, 0)),
        pl.BlockSpec((_BN, D), lambda i: (i, 0)),
        pl.BlockSpec((_BN, D), lambda i: (i, 0)),
        pl.BlockSpec((3 * D, D), lambda i: (0, 0)),
        pl.BlockSpec((1, D), lambda i: (0, 0)),
    ],
    out_specs=pl.BlockSpec((_BN, D), lambda i: (i, 0)),
    out_shape=jax.ShapeDtypeStruct((N, D), _F32),
)


@functools.partial(
    pl.kernel,
    out_type=(
        jax.ShapeDtypeStruct((N, D), _F32),
        jax.ShapeDtypeStruct((N, D), _F32),
    ),
    mesh=plsc.VectorSubcoreMesh(core_axis_name="c", subcore_axis_name="s"),
    scratch_types=[
        pltpu.VMEM((CHUNK,), jnp.int32),
        pltpu.VMEM((CHUNK,), jnp.int32),
        pltpu.VMEM((CHUNK,), jnp.int32),
        pltpu.VMEM((CHUNK,), jnp.int32),
        pltpu.VMEM((CHUNK, D), _F32),
        pltpu.VMEM((CHUNK, D), _F32),
        pltpu.VMEM((CHUNK, D), _F32),
        pltpu.VMEM((CHUNK, D), _F32),
        pltpu.VMEM((ZCHUNK, D), _F32),
        pltpu.VMEM_SHARED((N, D), _F32),
        pltpu.SemaphoreType.DMA,
        pltpu.SemaphoreType.DMA,
        pltpu.SemaphoreType.DMA,
        pltpu.SemaphoreType.DMA,
    ],
)
def _sc_aggregate(hx_s, we_s, send_s, recv_s, hx_a, we_a, send_a, recv_a,
                  z_s, z_a, sidx0, sidx1, ridx0, ridx1, hx0, hx1, we0, we1,
                  zbuf, z_sh, gsem0, gsem1, wsem0, wsem1):
    c = lax.axis_index("c")
    s = lax.axis_index("s")
    bufs = ((sidx0, ridx0, hx0, we0, gsem0, wsem0),
            (sidx1, ridx1, hx1, we1, gsem1, wsem1))

    zeros16 = jnp.zeros((16,), _F32)

    def _zrow(r, carry):
        for g in range(D // 16):
            zbuf[r, pl.ds(g * 16, 16)] = zeros16
        return carry

    lax.fori_loop(0, ZCHUNK, _zrow, 0)
    for j in range(NZ):
        pltpu.sync_copy(
            zbuf, z_sh.at[pl.ds(s * ROWS_PER_TILE + j * ZCHUNK, ZCHUNK)])

    @pl.when(s == 0)
    def _():
        pltpu.sync_copy(zbuf.at[pl.ds(0, TAIL_ROWS)],
                        z_sh.at[pl.ds(NUM_TILES * ROWS_PER_TILE, TAIL_ROWS)])

    plsc.subcore_barrier()

    def _process(hx_hbm, we_hbm, send_hbm, recv_hbm, z_hbm):
        def _start(k, b):
            si, ri, hxb, web, gsem, wsem = bufs[b]
            base = s * EDGES_PER_TILE + k * CHUNK
            pltpu.sync_copy(send_hbm.at[pl.ds(base, CHUNK)], si)
            pltpu.sync_copy(recv_hbm.at[pl.ds(base, CHUNK)], ri)
            pltpu.async_copy(hx_hbm.at[si], hxb, gsem)
            pltpu.async_copy(we_hbm.at[pl.ds(base, CHUNK)], web, wsem)

        def _finish(k, b):
            si, ri, hxb, web, gsem, wsem = bufs[b]
            base = s * EDGES_PER_TILE + k * CHUNK
            pltpu.make_async_copy(hx_hbm.at[si], hxb, gsem).wait()
            pltpu.make_async_copy(
                we_hbm.at[pl.ds(base, CHUNK)], web, wsem).wait()

            @plsc.parallel_loop(0, CHUNK, unroll=2)
            def _mul(r):
                for g in range(D // 16):
                    sl = pl.ds(g * 16, 16)
                    web[r, sl] = web[r, sl] * hxb[r, sl]

            pltpu.sync_copy(web, z_sh.at[ri], add=True)

        _start(0, 0)

        def _pair(i, carry):
            _start(2 * i + 1, 1)
            _finish(2 * i, 0)

            @pl.when(i < NPAIR - 1)
            def _():
                _start(2 * i + 2, 0)

            _finish(2 * i + 1, 1)
            return carry

        lax.fori_loop(0, NPAIR, _pair, 0)
        plsc.subcore_barrier()
        for j in range(NZ):
            sl = pl.ds(s * ROWS_PER_TILE + j * ZCHUNK, ZCHUNK)
            pltpu.sync_copy(z_sh.at[sl], z_hbm.at[sl])

        @pl.when(s == 0)
        def _():
            sl = pl.ds(NUM_TILES * ROWS_PER_TILE, TAIL_ROWS)
            pltpu.sync_copy(z_sh.at[sl], z_hbm.at[sl])

    @pl.when(c == 0)
    def _():
        _process(hx_s, we_s, send_s, recv_s, z_s)

    @pl.when(c == 1)
    def _():
        _process(hx_a, we_a, send_a, recv_a, z_a)


def kernel(x, feat_same, feat_anti, senders_same, receivers_same, senders_anti,
           receivers_anti, W_u_same, b_u_same, W_u_anti, b_u_anti, W_w_same,
           b_w_same, W_w_anti, b_w_anti, W_h_same, b_h_same, W_h_anti,
           b_h_anti, W_g, b_g):
    r = lambda b: b.reshape(1, D)
    hx_s, hx_a = _hx_call(x, W_h_same, r(b_h_same), W_h_anti, r(b_h_anti))
    we_s = _we_call(feat_same, W_u_same, r(b_u_same), W_w_same, r(b_w_same))
    we_a = _we_call(feat_anti, W_u_anti, r(b_u_anti), W_w_anti, r(b_w_anti))
    i32 = jnp.int32
    z_s, z_a = _sc_aggregate(
        hx_s, we_s, senders_same.astype(i32), receivers_same.astype(i32),
        hx_a, we_a, senders_anti.astype(i32), receivers_anti.astype(i32))
    return _upd_call(x, z_s, z_a, W_g, r(b_g))


# speedup vs baseline: 3.8763x; 1.0782x over previous
import functools

import jax
import jax.numpy as jnp
from jax import lax
from jax.experimental import pallas as pl
from jax.experimental.pallas import tpu as pltpu
from jax.experimental.pallas import tpu_sc as plsc

N = 10000
D = 128
E = 320000

NUM_TILES = 16
NUM_WORKERS = 32
EDGES_PER_WORKER = E // NUM_WORKERS
CHUNK = 80
NFULL = EDGES_PER_WORKER // CHUNK
NPAIR = (NFULL - 1) // 2
ROWS_PER_TILE = 624
ZCHUNK = 16
NZ = ROWS_PER_TILE // ZCHUNK
TAIL_ROWS = N - NUM_TILES * ROWS_PER_TILE

_F32 = jnp.float32


def _hx_body(x_ref, ws_ref, bs_ref, wa_ref, ba_ref, hs_ref, ha_ref):
    xv = x_ref[...]
    hs_ref[...] = jnp.tanh(
        jnp.dot(xv, ws_ref[...], preferred_element_type=_F32) + bs_ref[...])
    ha_ref[...] = jnp.tanh(
        jnp.dot(xv, wa_ref[...], preferred_element_type=_F32) + ba_ref[...])


_BN = 2000

_hx_call = pl.pallas_call(
    _hx_body,
    grid=(N // _BN,),
    in_specs=[
        pl.BlockSpec((_BN, D), lambda i: (i, 0)),
        pl.BlockSpec((D, D), lambda i: (0, 0)),
        pl.BlockSpec((1, D), lambda i: (0, 0)),
        pl.BlockSpec((D, D), lambda i: (0, 0)),
        pl.BlockSpec((1, D), lambda i: (0, 0)),
    ],
    out_specs=[
        pl.BlockSpec((_BN, D), lambda i: (i, 0)),
        pl.BlockSpec((_BN, D), lambda i: (i, 0)),
    ],
    out_shape=[
        jax.ShapeDtypeStruct((N, D), _F32),
        jax.ShapeDtypeStruct((N, D), _F32),
    ],
)


def _we_body(f_ref, wu_ref, bu_ref, ww_ref, bw_ref, o_ref):
    fb = f_ref[...]
    f = jnp.tanh(
        jnp.dot(fb, wu_ref[...], preferred_element_type=_F32) + bu_ref[...]) + fb
    o_ref[...] = jnp.tanh(
        jnp.dot(f, ww_ref[...], preferred_element_type=_F32) + bw_ref[...])


_BE = 5000

_we_call = pl.pallas_call(
    _we_body,
    grid=(E // _BE,),
    in_specs=[
        pl.BlockSpec((_BE, D), lambda i: (i, 0)),
        pl.BlockSpec((D, D), lambda i: (0, 0)),
        pl.BlockSpec((1, D), lambda i: (0, 0)),
        pl.BlockSpec((D, D), lambda i: (0, 0)),
        pl.BlockSpec((1, D), lambda i: (0, 0)),
    ],
    out_specs=pl.BlockSpec((_BE, D), lambda i: (i, 0)),
    out_shape=jax.ShapeDtypeStruct((E, D), _F32),
)


def _upd_body(x_ref, zs0_ref, zs1_ref, za0_ref, za1_ref, wg_ref, bg_ref,
              o_ref):
    xv = x_ref[...]
    acc = jnp.dot(xv, wg_ref[0:D, :], preferred_element_type=_F32)
    acc = acc + jnp.dot(zs0_ref[...] + zs1_ref[...], wg_ref[D:2 * D, :],
                        preferred_element_type=_F32)
    acc = acc + jnp.dot(za0_ref[...] + za1_ref[...], wg_ref[2 * D:3 * D, :],
                        preferred_element_type=_F32)
    o_ref[...] = xv + jnp.tanh(acc + bg_ref[...])


_upd_call = pl.pallas_call(
    _upd_body,
    grid=(N // _BN,),
    in_specs=[
        pl.BlockSpec((_BN, D), lambda i: (i, 0)),
        pl.BlockSpec((_BN, D), lambda i: (i, 0)),
        pl.BlockSpec((_BN, D), lambda i: (i, 0)),
        pl.BlockSpec((_BN, D), lambda i: (i, 0)),
        pl.BlockSpec((_BN, D), lambda i: (i, 0)),
        pl.BlockSpec((3 * D, D), lambda i: (0, 0)),
        pl.BlockSpec((1, D), lambda i: (0, 0)),
    ],
    out_specs=pl.BlockSpec((_BN, D), lambda i: (i, 0)),
    out_shape=jax.ShapeDtypeStruct((N, D), _F32),
)


@functools.partial(
    pl.kernel,
    out_type=(
        jax.ShapeDtypeStruct((N, D), _F32),
        jax.ShapeDtypeStruct((N, D), _F32),
    ),
    mesh=plsc.VectorSubcoreMesh(core_axis_name="c", subcore_axis_name="s"),
    scratch_types=[
        pltpu.VMEM((CHUNK,), jnp.int32),
        pltpu.VMEM((CHUNK,), jnp.int32),
        pltpu.VMEM((CHUNK,), jnp.int32),
        pltpu.VMEM((CHUNK,), jnp.int32),
        pltpu.VMEM((CHUNK, D), _F32),
        pltpu.VMEM((CHUNK, D), _F32),
        pltpu.VMEM((CHUNK, D), _F32),
        pltpu.VMEM((CHUNK, D), _F32),
        pltpu.VMEM((ZCHUNK, D), _F32),
        pltpu.VMEM_SHARED((N, D), _F32),
        pltpu.SemaphoreType.DMA,
        pltpu.SemaphoreType.DMA,
        pltpu.SemaphoreType.DMA,
        pltpu.SemaphoreType.DMA,
    ],
)
def _sc_aggregate(hx_hbm, we_hbm, send_hbm, recv_hbm,
                  z0_hbm, z1_hbm, sidx0, sidx1, ridx0, ridx1, hx0, hx1,
                  we0, we1, zbuf, z_sh, gsem0, gsem1, wsem0, wsem1):
    c = lax.axis_index("c")
    s = lax.axis_index("s")
    w = c * NUM_TILES + s
    bufs = ((sidx0, ridx0, hx0, we0, gsem0, wsem0),
            (sidx1, ridx1, hx1, we1, gsem1, wsem1))

    zeros16 = jnp.zeros((16,), _F32)

    def _zrow(r, carry):
        for g in range(D // 16):
            zbuf[r, pl.ds(g * 16, 16)] = zeros16
        return carry

    lax.fori_loop(0, ZCHUNK, _zrow, 0)
    for j in range(NZ):
        pltpu.sync_copy(
            zbuf, z_sh.at[pl.ds(s * ROWS_PER_TILE + j * ZCHUNK, ZCHUNK)])

    @pl.when(s == 0)
    def _():
        pltpu.sync_copy(zbuf.at[pl.ds(0, TAIL_ROWS)],
                        z_sh.at[pl.ds(NUM_TILES * ROWS_PER_TILE, TAIL_ROWS)])

    plsc.subcore_barrier()

    def _start(k, b):
        si, ri, hxb, web, gsem, wsem = bufs[b]
        base = w * EDGES_PER_WORKER + k * CHUNK
        pltpu.sync_copy(send_hbm.at[pl.ds(base, CHUNK)], si)
        pltpu.sync_copy(recv_hbm.at[pl.ds(base, CHUNK)], ri)
        pltpu.async_copy(hx_hbm.at[si], hxb, gsem)
        pltpu.async_copy(we_hbm.at[pl.ds(base, CHUNK)], web, wsem)

    def _finish(k, b):
        si, ri, hxb, web, gsem, wsem = bufs[b]
        base = w * EDGES_PER_WORKER + k * CHUNK
        pltpu.make_async_copy(hx_hbm.at[si], hxb, gsem).wait()
        pltpu.make_async_copy(
            we_hbm.at[pl.ds(base, CHUNK)], web, wsem).wait()

        @plsc.parallel_loop(0, CHUNK, unroll=2)
        def _mul(r):
            for g in range(D // 16):
                sl = pl.ds(g * 16, 16)
                web[r, sl] = web[r, sl] * hxb[r, sl]

        pltpu.sync_copy(web, z_sh.at[ri], add=True)

    _start(0, 0)

    def _pair(i, carry):
        _start(2 * i + 1, 1)
        _finish(2 * i, 0)
        _start(2 * i + 2, 0)
        _finish(2 * i + 1, 1)
        return carry

    lax.fori_loop(0, NPAIR, _pair, 0)
    _finish(NFULL - 1, 0)
    plsc.subcore_barrier()

    def _writeout(z_out):
        for j in range(NZ):
            sl = pl.ds(s * ROWS_PER_TILE + j * ZCHUNK, ZCHUNK)
            pltpu.sync_copy(z_sh.at[sl], z_out.at[sl])

        @pl.when(s == 0)
        def _():
            sl = pl.ds(NUM_TILES * ROWS_PER_TILE, TAIL_ROWS)
            pltpu.sync_copy(z_sh.at[sl], z_out.at[sl])

    @pl.when(c == 0)
    def _():
        _writeout(z0_hbm)

    @pl.when(c == 1)
    def _():
        _writeout(z1_hbm)


def kernel(x, feat_same, feat_anti, senders_same, receivers_same, senders_anti,
           receivers_anti, W_u_same, b_u_same, W_u_anti, b_u_anti, W_w_same,
           b_w_same, W_w_anti, b_w_anti, W_h_same, b_h_same, W_h_anti,
           b_h_anti, W_g, b_g):
    r = lambda b: b.reshape(1, D)
    i32 = jnp.int32
    hx_s, hx_a = _hx_call(x, W_h_same, r(b_h_same), W_h_anti, r(b_h_anti))
    we_s = _we_call(feat_same, W_u_same, r(b_u_same), W_w_same, r(b_w_same))
    zs0, zs1 = _sc_aggregate(
        hx_s, we_s, senders_same.astype(i32), receivers_same.astype(i32))
    we_a = _we_call(feat_anti, W_u_anti, r(b_u_anti), W_w_anti, r(b_w_anti))
    za0, za1 = _sc_aggregate(
        hx_a, we_a, senders_anti.astype(i32), receivers_anti.astype(i32))
    return _upd_call(x, zs0, zs1, za0, za1, W_g, r(b_g))


# speedup vs baseline: 3.8788x; 1.0007x over previous
import functools

import jax
import jax.numpy as jnp
from jax import lax
from jax.experimental import pallas as pl
from jax.experimental.pallas import tpu as pltpu
from jax.experimental.pallas import tpu_sc as plsc

N = 10000
D = 128
E = 320000

NUM_TILES = 16
NUM_WORKERS = 32
EDGES_PER_WORKER = E // NUM_WORKERS
CHUNK = 80
NFULL = EDGES_PER_WORKER // CHUNK
NPAIR = (NFULL - 1) // 2
ROWS_PER_TILE = 624
ZCHUNK = 16
NZ = ROWS_PER_TILE // ZCHUNK
TAIL_ROWS = N - NUM_TILES * ROWS_PER_TILE

_F32 = jnp.float32


def _hx_body(x_ref, ws_ref, bs_ref, wa_ref, ba_ref, hs_ref, ha_ref):
    xv = x_ref[...]
    hs_ref[...] = jnp.tanh(
        jnp.dot(xv, ws_ref[...], preferred_element_type=_F32) + bs_ref[...])
    ha_ref[...] = jnp.tanh(
        jnp.dot(xv, wa_ref[...], preferred_element_type=_F32) + ba_ref[...])


_BN = 2000

_hx_call = pl.pallas_call(
    _hx_body,
    grid=(N // _BN,),
    in_specs=[
        pl.BlockSpec((_BN, D), lambda i: (i, 0)),
        pl.BlockSpec((D, D), lambda i: (0, 0)),
        pl.BlockSpec((1, D), lambda i: (0, 0)),
        pl.BlockSpec((D, D), lambda i: (0, 0)),
        pl.BlockSpec((1, D), lambda i: (0, 0)),
    ],
    out_specs=[
        pl.BlockSpec((_BN, D), lambda i: (i, 0)),
        pl.BlockSpec((_BN, D), lambda i: (i, 0)),
    ],
    out_shape=[
        jax.ShapeDtypeStruct((N, D), _F32),
        jax.ShapeDtypeStruct((N, D), _F32),
    ],
)


def _we_body(f_ref, wu_ref, bu_ref, ww_ref, bw_ref, o_ref):
    fb = f_ref[...]
    f = jnp.tanh(
        jnp.dot(fb, wu_ref[...], preferred_element_type=_F32) + bu_ref[...]) + fb
    o_ref[...] = jnp.tanh(
        jnp.dot(f, ww_ref[...], preferred_element_type=_F32) + bw_ref[...])


_BE = 5000

_we_call = pl.pallas_call(
    _we_body,
    grid=(E // _BE,),
    in_specs=[
        pl.BlockSpec((_BE, D), lambda i: (i, 0)),
        pl.BlockSpec((D, D), lambda i: (0, 0)),
        pl.BlockSpec((1, D), lambda i: (0, 0)),
        pl.BlockSpec((D, D), lambda i: (0, 0)),
        pl.BlockSpec((1, D), lambda i: (0, 0)),
    ],
    out_specs=pl.BlockSpec((_BE, D), lambda i: (i, 0)),
    out_shape=jax.ShapeDtypeStruct((E, D), _F32),
)


def _upd_body(x_ref, zs0_ref, zs1_ref, za0_ref, za1_ref, wg_ref, bg_ref,
              o_ref):
    xv = x_ref[...]
    acc = jnp.dot(xv, wg_ref[0:D, :], preferred_element_type=_F32)
    acc = acc + jnp.dot(zs0_ref[...] + zs1_ref[...], wg_ref[D:2 * D, :],
                        preferred_element_type=_F32)
    acc = acc + jnp.dot(za0_ref[...] + za1_ref[...], wg_ref[2 * D:3 * D, :],
                        preferred_element_type=_F32)
    o_ref[...] = xv + jnp.tanh(acc + bg_ref[...])


_upd_call = pl.pallas_call(
    _upd_body,
    grid=(N // _BN,),
    in_specs=[
        pl.BlockSpec((_BN, D), lambda i: (i, 0)),
        pl.BlockSpec((_BN, D), lambda i: (i, 0)),
        pl.BlockSpec((_BN, D), lambda i: (i, 0)),
        pl.BlockSpec((_BN, D), lambda i: (i, 0)),
        pl.BlockSpec((_BN, D), lambda i: (i, 0)),
        pl.BlockSpec((3 * D, D), lambda i: (0, 0)),
        pl.BlockSpec((1, D), lambda i: (0, 0)),
    ],
    out_specs=pl.BlockSpec((_BN, D), lambda i: (i, 0)),
    out_shape=jax.ShapeDtypeStruct((N, D), _F32),
)


@functools.partial(
    pl.kernel,
    out_type=(
        jax.ShapeDtypeStruct((N, D), _F32),
        jax.ShapeDtypeStruct((N, D), _F32),
    ),
    mesh=plsc.VectorSubcoreMesh(core_axis_name="c", subcore_axis_name="s"),
    scratch_types=[
        pltpu.VMEM((CHUNK,), jnp.int32),
        pltpu.VMEM((CHUNK,), jnp.int32),
        pltpu.VMEM((CHUNK,), jnp.int32),
        pltpu.VMEM((CHUNK,), jnp.int32),
        pltpu.VMEM((CHUNK, D), _F32),
        pltpu.VMEM((CHUNK, D), _F32),
        pltpu.VMEM((CHUNK, D), _F32),
        pltpu.VMEM((CHUNK, D), _F32),
        pltpu.VMEM((ZCHUNK, D), _F32),
        pltpu.VMEM_SHARED((N, D), _F32),
        pltpu.SemaphoreType.DMA,
        pltpu.SemaphoreType.DMA,
        pltpu.SemaphoreType.DMA,
        pltpu.SemaphoreType.DMA,
        pltpu.SemaphoreType.DMA,
        pltpu.SemaphoreType.DMA,
    ],
)
def _sc_aggregate(hx_hbm, we_hbm, send_hbm, recv_hbm,
                  z0_hbm, z1_hbm, sidx0, sidx1, ridx0, ridx1, hx0, hx1,
                  we0, we1, zbuf, z_sh, gsem0, gsem1, wsem0, wsem1,
                  ssem0, ssem1):
    c = lax.axis_index("c")
    s = lax.axis_index("s")
    w = c * NUM_TILES + s
    bufs = ((sidx0, ridx0, hx0, we0, gsem0, wsem0, ssem0),
            (sidx1, ridx1, hx1, we1, gsem1, wsem1, ssem1))

    zeros16 = jnp.zeros((16,), _F32)

    def _zrow(r, carry):
        for g in range(D // 16):
            zbuf[r, pl.ds(g * 16, 16)] = zeros16
        return carry

    lax.fori_loop(0, ZCHUNK, _zrow, 0)
    for j in range(NZ):
        pltpu.sync_copy(
            zbuf, z_sh.at[pl.ds(s * ROWS_PER_TILE + j * ZCHUNK, ZCHUNK)])

    @pl.when(s == 0)
    def _():
        pltpu.sync_copy(zbuf.at[pl.ds(0, TAIL_ROWS)],
                        z_sh.at[pl.ds(NUM_TILES * ROWS_PER_TILE, TAIL_ROWS)])

    plsc.subcore_barrier()

    def _start(k, b):
        si, ri, hxb, web, gsem, wsem, ssem = bufs[b]
        base = w * EDGES_PER_WORKER + k * CHUNK

        @pl.when(k >= 2)
        def _():
            pltpu.make_async_copy(web, z_sh.at[ri], ssem).wait()

        pltpu.sync_copy(send_hbm.at[pl.ds(base, CHUNK)], si)
        pltpu.sync_copy(recv_hbm.at[pl.ds(base, CHUNK)], ri)
        pltpu.async_copy(hx_hbm.at[si], hxb, gsem)
        pltpu.async_copy(we_hbm.at[pl.ds(base, CHUNK)], web, wsem)

    def _finish(k, b):
        si, ri, hxb, web, gsem, wsem, ssem = bufs[b]
        base = w * EDGES_PER_WORKER + k * CHUNK
        pltpu.make_async_copy(hx_hbm.at[si], hxb, gsem).wait()
        pltpu.make_async_copy(
            we_hbm.at[pl.ds(base, CHUNK)], web, wsem).wait()

        @plsc.parallel_loop(0, CHUNK, unroll=2)
        def _mul(r):
            for g in range(D // 16):
                sl = pl.ds(g * 16, 16)
                web[r, sl] = web[r, sl] * hxb[r, sl]

        pltpu.async_copy(web, z_sh.at[ri], ssem, add=True)

    _start(0, 0)

    def _pair(i, carry):
        _start(2 * i + 1, 1)
        _finish(2 * i, 0)
        _start(2 * i + 2, 0)
        _finish(2 * i + 1, 1)
        return carry

    lax.fori_loop(0, NPAIR, _pair, 0)
    _finish(NFULL - 1, 0)
    pltpu.make_async_copy(we1, z_sh.at[ridx1], ssem1).wait()
    pltpu.make_async_copy(we0, z_sh.at[ridx0], ssem0).wait()
    plsc.subcore_barrier()

    def _writeout(z_out):
        for j in range(NZ):
            sl = pl.ds(s * ROWS_PER_TILE + j * ZCHUNK, ZCHUNK)
            pltpu.sync_copy(z_sh.at[sl], z_out.at[sl])

        @pl.when(s == 0)
        def _():
            sl = pl.ds(NUM_TILES * ROWS_PER_TILE, TAIL_ROWS)
            pltpu.sync_copy(z_sh.at[sl], z_out.at[sl])

    @pl.when(c == 0)
    def _():
        _writeout(z0_hbm)

    @pl.when(c == 1)
    def _():
        _writeout(z1_hbm)


def kernel(x, feat_same, feat_anti, senders_same, receivers_same, senders_anti,
           receivers_anti, W_u_same, b_u_same, W_u_anti, b_u_anti, W_w_same,
           b_w_same, W_w_anti, b_w_anti, W_h_same, b_h_same, W_h_anti,
           b_h_anti, W_g, b_g):
    r = lambda b: b.reshape(1, D)
    i32 = jnp.int32
    hx_s, hx_a = _hx_call(x, W_h_same, r(b_h_same), W_h_anti, r(b_h_anti))
    we_s = _we_call(feat_same, W_u_same, r(b_u_same), W_w_same, r(b_w_same))
    zs0, zs1 = _sc_aggregate(
        hx_s, we_s, senders_same.astype(i32), receivers_same.astype(i32))
    we_a = _we_call(feat_anti, W_u_anti, r(b_u_anti), W_w_anti, r(b_w_anti))
    za0, za1 = _sc_aggregate(
        hx_a, we_a, senders_anti.astype(i32), receivers_anti.astype(i32))
    return _upd_call(x, zs0, zs1, za0, za1, W_g, r(b_g))


# speedup vs baseline: 4.7662x; 1.2288x over previous
import functools

import jax
import jax.numpy as jnp
from jax import lax
from jax.experimental import pallas as pl
from jax.experimental.pallas import tpu as pltpu
from jax.experimental.pallas import tpu_sc as plsc

N = 10000
D = 128
E = 320000

NUM_TILES = 16
NUM_WORKERS = 32
EDGES_PER_WORKER = E // NUM_WORKERS
CHUNK = 80
NFULL = EDGES_PER_WORKER // CHUNK
NPAIR = (NFULL - 1) // 2
ROWS_PER_TILE = 624
ZCHUNK = 16
NZ = ROWS_PER_TILE // ZCHUNK
TAIL_ROWS = N - NUM_TILES * ROWS_PER_TILE

_F32 = jnp.float32


def _hx_body(x_ref, ws_ref, bs_ref, wa_ref, ba_ref, hs_ref, ha_ref):
    xv = x_ref[...]
    hs_ref[...] = jnp.tanh(
        jnp.dot(xv, ws_ref[...], preferred_element_type=_F32) + bs_ref[...])
    ha_ref[...] = jnp.tanh(
        jnp.dot(xv, wa_ref[...], preferred_element_type=_F32) + ba_ref[...])


_BN = 2000

_hx_call = pl.pallas_call(
    _hx_body,
    grid=(N // _BN,),
    in_specs=[
        pl.BlockSpec((_BN, D), lambda i: (i, 0)),
        pl.BlockSpec((D, D), lambda i: (0, 0)),
        pl.BlockSpec((1, D), lambda i: (0, 0)),
        pl.BlockSpec((D, D), lambda i: (0, 0)),
        pl.BlockSpec((1, D), lambda i: (0, 0)),
    ],
    out_specs=[
        pl.BlockSpec((_BN, D), lambda i: (i, 0)),
        pl.BlockSpec((_BN, D), lambda i: (i, 0)),
    ],
    out_shape=[
        jax.ShapeDtypeStruct((N, D), _F32),
        jax.ShapeDtypeStruct((N, D), _F32),
    ],
)


def _we_body(f_ref, wu_ref, bu_ref, ww_ref, bw_ref, o_ref):
    fb = f_ref[...]
    f = jnp.tanh(
        jnp.dot(fb, wu_ref[...], preferred_element_type=_F32) + bu_ref[...]) + fb
    o_ref[...] = jnp.tanh(
        jnp.dot(f, ww_ref[...], preferred_element_type=_F32) + bw_ref[...])


_BE = 5000

_we_call = pl.pallas_call(
    _we_body,
    grid=(E // _BE,),
    in_specs=[
        pl.BlockSpec((_BE, D), lambda i: (i, 0)),
        pl.BlockSpec((D, D), lambda i: (0, 0)),
        pl.BlockSpec((1, D), lambda i: (0, 0)),
        pl.BlockSpec((D, D), lambda i: (0, 0)),
        pl.BlockSpec((1, D), lambda i: (0, 0)),
    ],
    out_specs=pl.BlockSpec((_BE, D), lambda i: (i, 0)),
    out_shape=jax.ShapeDtypeStruct((E, D), _F32),
)


def _upd_body(x_ref, zs0_ref, zs1_ref, za0_ref, za1_ref, wg_ref, bg_ref,
              o_ref):
    xv = x_ref[...]
    acc = jnp.dot(xv, wg_ref[0:D, :], preferred_element_type=_F32)
    acc = acc + jnp.dot(zs0_ref[...] + zs1_ref[...], wg_ref[D:2 * D, :],
                        preferred_element_type=_F32)
    acc = acc + jnp.dot(za0_ref[...] + za1_ref[...], wg_ref[2 * D:3 * D, :],
                        preferred_element_type=_F32)
    o_ref[...] = xv + jnp.tanh(acc + bg_ref[...])


_upd_call = pl.pallas_call(
    _upd_body,
    grid=(N // _BN,),
    in_specs=[
        pl.BlockSpec((_BN, D), lambda i: (i, 0)),
        pl.BlockSpec((_BN, D), lambda i: (i, 0)),
        pl.BlockSpec((_BN, D), lambda i: (i, 0)),
        pl.BlockSpec((_BN, D), lambda i: (i, 0)),
        pl.BlockSpec((_BN, D), lambda i: (i, 0)),
        pl.BlockSpec((3 * D, D), lambda i: (0, 0)),
        pl.BlockSpec((1, D), lambda i: (0, 0)),
    ],
    out_specs=pl.BlockSpec((_BN, D), lambda i: (i, 0)),
    out_shape=jax.ShapeDtypeStruct((N, D), _F32),
)


@functools.partial(
    pl.kernel,
    out_type=(
        jax.ShapeDtypeStruct((N, D), _F32),
        jax.ShapeDtypeStruct((N, D), _F32),
    ),
    mesh=plsc.VectorSubcoreMesh(core_axis_name="c", subcore_axis_name="s"),
    scratch_types=[
        pltpu.VMEM((CHUNK,), jnp.int32),
        pltpu.VMEM((CHUNK,), jnp.int32),
        pltpu.VMEM((CHUNK,), jnp.int32),
        pltpu.VMEM((CHUNK,), jnp.int32),
        pltpu.VMEM((CHUNK, D), _F32),
        pltpu.VMEM((CHUNK, D), _F32),
        pltpu.VMEM((CHUNK, D), _F32),
        pltpu.VMEM((CHUNK, D), _F32),
        pltpu.VMEM((ZCHUNK, D), _F32),
        pltpu.VMEM_SHARED((N, D), _F32),
        pltpu.SemaphoreType.DMA,
        pltpu.SemaphoreType.DMA,
        pltpu.SemaphoreType.DMA,
        pltpu.SemaphoreType.DMA,
        pltpu.SemaphoreType.DMA,
        pltpu.SemaphoreType.DMA,
        pltpu.SemaphoreType.DMA,
        pltpu.SemaphoreType.DMA,
        pltpu.SemaphoreType.DMA,
        pltpu.SemaphoreType.DMA,
    ],
)
def _sc_aggregate(hx_hbm, we_hbm, send_hbm, recv_hbm,
                  z0_hbm, z1_hbm, sidx0, sidx1, ridx0, ridx1, hx0, hx1,
                  we0, we1, zbuf, z_sh, gsem0, gsem1, wsem0, wsem1,
                  ssem0, ssem1, isem0, isem1, rsem0, rsem1):
    c = lax.axis_index("c")
    s = lax.axis_index("s")
    w = c * NUM_TILES + s
    bufs = ((sidx0, ridx0, hx0, we0, gsem0, wsem0, ssem0, isem0, rsem0),
            (sidx1, ridx1, hx1, we1, gsem1, wsem1, ssem1, isem1, rsem1))

    zeros16 = jnp.zeros((16,), _F32)

    def _zrow(r, carry):
        for g in range(D // 16):
            zbuf[r, pl.ds(g * 16, 16)] = zeros16
        return carry

    lax.fori_loop(0, ZCHUNK, _zrow, 0)
    for j in range(NZ):
        pltpu.sync_copy(
            zbuf, z_sh.at[pl.ds(s * ROWS_PER_TILE + j * ZCHUNK, ZCHUNK)])

    @pl.when(s == 0)
    def _():
        pltpu.sync_copy(zbuf.at[pl.ds(0, TAIL_ROWS)],
                        z_sh.at[pl.ds(NUM_TILES * ROWS_PER_TILE, TAIL_ROWS)])

    plsc.subcore_barrier()

    def _base(k):
        return w * EDGES_PER_WORKER + k * CHUNK

    def _start(k, b):
        si, ri, hxb, web, gsem, wsem, ssem, isem, rsem = bufs[b]
        base = _base(k)

        @pl.when(k >= 2)
        def _():
            pltpu.make_async_copy(web, z_sh.at[ri], ssem).wait()

        pltpu.async_copy(recv_hbm.at[pl.ds(base, CHUNK)], ri, rsem)
        pltpu.make_async_copy(send_hbm.at[pl.ds(base, CHUNK)], si, isem).wait()
        pltpu.async_copy(hx_hbm.at[si], hxb, gsem)
        pltpu.async_copy(we_hbm.at[pl.ds(base, CHUNK)], web, wsem)

    def _finish(k, b):
        si, ri, hxb, web, gsem, wsem, ssem, isem, rsem = bufs[b]
        base = _base(k)
        pltpu.make_async_copy(hx_hbm.at[si], hxb, gsem).wait()
        pltpu.make_async_copy(
            we_hbm.at[pl.ds(base, CHUNK)], web, wsem).wait()

        @pl.when(k < NFULL - 2)
        def _():
            pltpu.async_copy(
                send_hbm.at[pl.ds(_base(k + 2), CHUNK)], si, isem)

        @plsc.parallel_loop(0, CHUNK, unroll=4)
        def _mul(r):
            for g in range(D // 16):
                sl = pl.ds(g * 16, 16)
                web[r, sl] = web[r, sl] * hxb[r, sl]

        pltpu.make_async_copy(
            recv_hbm.at[pl.ds(base, CHUNK)], ri, rsem).wait()
        pltpu.async_copy(web, z_sh.at[ri], ssem, add=True)

    pltpu.async_copy(send_hbm.at[pl.ds(_base(0), CHUNK)], sidx0, isem0)
    pltpu.async_copy(send_hbm.at[pl.ds(_base(1), CHUNK)], sidx1, isem1)
    _start(0, 0)

    def _pair(i, carry):
        _start(2 * i + 1, 1)
        _finish(2 * i, 0)
        _start(2 * i + 2, 0)
        _finish(2 * i + 1, 1)
        return carry

    lax.fori_loop(0, NPAIR, _pair, 0)
    _finish(NFULL - 1, 0)
    pltpu.make_async_copy(we1, z_sh.at[ridx1], ssem1).wait()
    pltpu.make_async_copy(we0, z_sh.at[ridx0], ssem0).wait()
    plsc.subcore_barrier()

    def _writeout(z_out):
        for j in range(NZ):
            sl = pl.ds(s * ROWS_PER_TILE + j * ZCHUNK, ZCHUNK)
            pltpu.sync_copy(z_sh.at[sl], z_out.at[sl])

        @pl.when(s == 0)
        def _():
            sl = pl.ds(NUM_TILES * ROWS_PER_TILE, TAIL_ROWS)
            pltpu.sync_copy(z_sh.at[sl], z_out.at[sl])

    @pl.when(c == 0)
    def _():
        _writeout(z0_hbm)

    @pl.when(c == 1)
    def _():
        _writeout(z1_hbm)


def kernel(x, feat_same, feat_anti, senders_same, receivers_same, senders_anti,
           receivers_anti, W_u_same, b_u_same, W_u_anti, b_u_anti, W_w_same,
           b_w_same, W_w_anti, b_w_anti, W_h_same, b_h_same, W_h_anti,
           b_h_anti, W_g, b_g):
    r = lambda b: b.reshape(1, D)
    i32 = jnp.int32
    hx_s, hx_a = _hx_call(x, W_h_same, r(b_h_same), W_h_anti, r(b_h_anti))
    we_s = _we_call(feat_same, W_u_same, r(b_u_same), W_w_same, r(b_w_same))
    zs0, zs1 = _sc_aggregate(
        hx_s, we_s, senders_same.astype(i32), receivers_same.astype(i32))
    we_a = _we_call(feat_anti, W_u_anti, r(b_u_anti), W_w_anti, r(b_w_anti))
    za0, za1 = _sc_aggregate(
        hx_a, we_a, senders_anti.astype(i32), receivers_anti.astype(i32))
    return _upd_call(x, zs0, zs1, za0, za1, W_g, r(b_g))
